# baseline (speedup 1.0000x reference)
import functools

import jax
import jax.numpy as jnp
from jax import lax
from jax.experimental import pallas as pl
from jax.experimental.pallas import tpu as pltpu
from jax.experimental.pallas import tpu_sc as plsc

VOCAB = 1000000
VE_DIM = 64
MODEL_DIM = 128
B = 4096
L = 200
N = B * L
HN = N // 2

NC = 2
NS = 16
NW = NC * NS
NSPLIT = 5
HC = HN // NSPLIT
SPAN = HC // NW
CHUNK = 128
N_CHUNKS = SPAN // CHUNK
NBUF = 4

TBLK = 8192
HBLK = TBLK // 2
T_GRID = (VOCAB + TBLK - 1) // TBLK
VPAD = T_GRID * TBLK

R_PAIR = 8192
N_BLOCKS = HN // R_PAIR


def _tr_body(x_ref, o_ref):
    eye = jnp.eye(VE_DIM, dtype=jnp.float32)
    o_ref[:, :VE_DIM] = jnp.dot(
        x_ref[:, :HBLK].T, eye, preferred_element_type=jnp.float32
    )
    o_ref[:, VE_DIM:] = jnp.dot(
        x_ref[:, HBLK:].T, eye, preferred_element_type=jnp.float32
    )


def _tc_transpose(wt):
    return pl.pallas_call(
        _tr_body,
        grid=(T_GRID,),
        in_specs=[pl.BlockSpec((VE_DIM, TBLK), lambda i: (0, i))],
        out_specs=pl.BlockSpec((HBLK, 2 * VE_DIM), lambda i: (i, 0)),
        out_shape=jax.ShapeDtypeStruct((T_GRID * HBLK, 2 * VE_DIM), jnp.float32),
        compiler_params=pltpu.CompilerParams(fuse_transposed_lhs_in_matmul=True),
    )(wt)


def _sc_gather_body(coff, idx_hbm, table_hbm, out_hbm, idx_v, bufa, bufb,
                    sema, semb):
    c = lax.axis_index("c")
    s = lax.axis_index("s")
    wid = s * NC + c

    pltpu.sync_copy(idx_hbm.at[pl.ds(coff + wid * SPAN, SPAN)], idx_v.at[0])
    pltpu.sync_copy(idx_hbm.at[pl.ds(HN + coff + wid * SPAN, SPAN)],
                    idx_v.at[1])

    def _fill(j, b):
        pltpu.async_copy(
            table_hbm.at[idx_v.at[0, pl.ds(j * CHUNK, CHUNK)]], bufa.at[b],
            sema.at[b],
        )
        pltpu.async_copy(
            table_hbm.at[idx_v.at[1, pl.ds(j * CHUNK, CHUNK)]], bufb.at[b],
            semb.at[b],
        )

    for b in range(NBUF):
        _fill(b, b)

    row0 = wid * SPAN

    def outer(j0, carry):
        for b in range(NBUF):
            j = j0 * NBUF + b
            pltpu.make_async_copy(
                table_hbm.at[idx_v.at[0, pl.ds(j * CHUNK, CHUNK)]], bufa.at[b],
                sema.at[b],
            ).wait()
            pltpu.make_async_copy(
                table_hbm.at[idx_v.at[1, pl.ds(j * CHUNK, CHUNK)]], bufb.at[b],
                semb.at[b],
            ).wait()
            r = row0 + j * CHUNK
            pltpu.sync_copy(bufa.at[b],
                            out_hbm.at[pl.ds(r, CHUNK), pl.ds(0, VE_DIM)])
            pltpu.sync_copy(bufb.at[b],
                            out_hbm.at[pl.ds(r, CHUNK), pl.ds(VE_DIM, VE_DIM)])

            @pl.when(j + NBUF < N_CHUNKS)
            def _refill(b=b, j=j):
                _fill(j + NBUF, b)

        return carry

    lax.fori_loop(0, N_CHUNKS // NBUF, outer, None)


def _sc_gather(idx, table, split):
    mesh = plsc.VectorSubcoreMesh(core_axis_name="c", subcore_axis_name="s")
    return pl.kernel(
        functools.partial(_sc_gather_body, split * HC),
        out_type=jax.ShapeDtypeStruct((HC, 2 * VE_DIM), jnp.float32),
        name=f"sc_pair_gather_{split}",
        mesh=mesh,
        scratch_types=[
            pltpu.VMEM((2, SPAN), jnp.int32),
            pltpu.VMEM((NBUF, CHUNK, VE_DIM), jnp.float32),
            pltpu.VMEM((NBUF, CHUNK, VE_DIM), jnp.float32),
            pltpu.SemaphoreType.DMA((NBUF,)),
            pltpu.SemaphoreType.DMA((NBUF,)),
        ],
        compiler_params=pltpu.CompilerParams(use_tc_tiling_on_sc=False),
    )(idx, table)


def _mm_compute(x_ref, w_ref, s_ref, o_ref):
    sc = s_ref[0]
    w = w_ref[...]
    o_ref[0] = (
        jnp.dot(x_ref[:, :VE_DIM], w, preferred_element_type=jnp.float32) * sc
    )
    o_ref[1] = (
        jnp.dot(x_ref[:, VE_DIM:], w, preferred_element_type=jnp.float32) * sc
    )


def _mm_body_first(x_ref, w_ref, s_ref, o_ref):
    _mm_compute(x_ref, w_ref, s_ref, o_ref)


def _mm_body_acc(acc_ref, x_ref, w_ref, s_ref, o_ref):
    del acc_ref
    _mm_compute(x_ref, w_ref, s_ref, o_ref)


C_BLOCKS = HC // R_PAIR


def _tc_project_split(acc, x, w_t, scale, split):
    base = split * C_BLOCKS
    x_spec = pl.BlockSpec((R_PAIR, 2 * VE_DIM), lambda i: (i, 0))
    w_spec = pl.BlockSpec((VE_DIM, MODEL_DIM), lambda i: (0, 0))
    s_spec = pl.BlockSpec(memory_space=pltpu.SMEM)
    out_spec = pl.BlockSpec(
        (2, R_PAIR, MODEL_DIM), lambda i: (0, i + base, 0)
    )
    out_shape = jax.ShapeDtypeStruct((2, HN, MODEL_DIM), jnp.float32)
    if acc is None:
        return pl.pallas_call(
            _mm_body_first,
            grid=(C_BLOCKS,),
            in_specs=[x_spec, w_spec, s_spec],
            out_specs=out_spec,
            out_shape=out_shape,
        )(x, w_t, scale)
    return pl.pallas_call(
        _mm_body_acc,
        grid=(C_BLOCKS,),
        in_specs=[pl.BlockSpec(memory_space=pl.ANY), x_spec, w_spec, s_spec],
        out_specs=out_spec,
        out_shape=out_shape,
        input_output_aliases={0: 0},
    )(acc, x, w_t, scale)


def kernel(token_ids, embed_weight, proj_weight, scale):
    ids = token_ids.astype(jnp.int32).reshape(-1)
    ids = (ids & ~(TBLK - 1)) + 2 * (ids & (HBLK - 1)) + ((ids >> 12) & 1)
    table_lin = _tc_transpose(embed_weight.T).reshape(VPAD, VE_DIM)
    w_t = proj_weight.T
    sc = scale.reshape(1)
    out = None
    for split in range(NSPLIT):
        paired = _sc_gather(ids, table_lin, split)
        out = _tc_project_split(out, paired, w_t, sc, split)
    return out.reshape(B, L, MODEL_DIM)


# speedup vs baseline: 1.0576x; 1.0576x over previous
import functools

import jax
import jax.numpy as jnp
from jax import lax
from jax.experimental import pallas as pl
from jax.experimental.pallas import tpu as pltpu
from jax.experimental.pallas import tpu_sc as plsc

VOCAB = 1000000
VE_DIM = 64
MODEL_DIM = 128
B = 4096
L = 200
N = B * L
HN = N // 2

NC = 2
NS = 16
NW = NC * NS
NSPLIT = 5
HC = HN // NSPLIT
SPAN = HC // NW
CHUNK = 128
N_CHUNKS = SPAN // CHUNK
NBUF = 4

TBLK = 16384
HBLK = TBLK // 2
HSHIFT = 13
T_GRID = (VOCAB + TBLK - 1) // TBLK
VPAD = T_GRID * TBLK

R_PAIR = 8192
N_BLOCKS = HN // R_PAIR


def _tr_body(x_ref, o_ref):
    eye = jnp.eye(VE_DIM, dtype=jnp.float32)
    o_ref[:, :VE_DIM] = jnp.dot(
        x_ref[:, :HBLK].T, eye, preferred_element_type=jnp.float32
    )
    o_ref[:, VE_DIM:] = jnp.dot(
        x_ref[:, HBLK:].T, eye, preferred_element_type=jnp.float32
    )


def _tc_transpose(wt):
    return pl.pallas_call(
        _tr_body,
        grid=(T_GRID,),
        in_specs=[pl.BlockSpec((VE_DIM, TBLK), lambda i: (0, i))],
        out_specs=pl.BlockSpec((HBLK, 2 * VE_DIM), lambda i: (i, 0)),
        out_shape=jax.ShapeDtypeStruct((T_GRID * HBLK, 2 * VE_DIM), jnp.float32),
        compiler_params=pltpu.CompilerParams(fuse_transposed_lhs_in_matmul=True),
    )(wt)


def _sc_gather_body(coff, idx_hbm, table_hbm, out_hbm, idx_v, bufa, bufb,
                    sema, semb):
    c = lax.axis_index("c")
    s = lax.axis_index("s")
    wid = s * NC + c

    pltpu.sync_copy(idx_hbm.at[pl.ds(coff + wid * SPAN, SPAN)], idx_v.at[0])
    pltpu.sync_copy(idx_hbm.at[pl.ds(HN + coff + wid * SPAN, SPAN)],
                    idx_v.at[1])

    def _fill(j, b):
        pltpu.async_copy(
            table_hbm.at[idx_v.at[0, pl.ds(j * CHUNK, CHUNK)]], bufa.at[b],
            sema.at[b],
        )
        pltpu.async_copy(
            table_hbm.at[idx_v.at[1, pl.ds(j * CHUNK, CHUNK)]], bufb.at[b],
            semb.at[b],
        )

    for b in range(NBUF):
        _fill(b, b)

    row0 = wid * SPAN

    def outer(j0, carry):
        for b in range(NBUF):
            j = j0 * NBUF + b
            pltpu.make_async_copy(
                table_hbm.at[idx_v.at[0, pl.ds(j * CHUNK, CHUNK)]], bufa.at[b],
                sema.at[b],
            ).wait()
            pltpu.make_async_copy(
                table_hbm.at[idx_v.at[1, pl.ds(j * CHUNK, CHUNK)]], bufb.at[b],
                semb.at[b],
            ).wait()
            r = row0 + j * CHUNK
            pltpu.sync_copy(bufa.at[b],
                            out_hbm.at[pl.ds(r, CHUNK), pl.ds(0, VE_DIM)])
            pltpu.sync_copy(bufb.at[b],
                            out_hbm.at[pl.ds(r, CHUNK), pl.ds(VE_DIM, VE_DIM)])

            @pl.when(j + NBUF < N_CHUNKS)
            def _refill(b=b, j=j):
                _fill(j + NBUF, b)

        return carry

    lax.fori_loop(0, N_CHUNKS // NBUF, outer, None)


def _sc_gather(idx, table, split):
    mesh = plsc.VectorSubcoreMesh(core_axis_name="c", subcore_axis_name="s")
    return pl.kernel(
        functools.partial(_sc_gather_body, split * HC),
        out_type=jax.ShapeDtypeStruct((HC, 2 * VE_DIM), jnp.float32),
        name=f"sc_pair_gather_{split}",
        mesh=mesh,
        scratch_types=[
            pltpu.VMEM((2, SPAN), jnp.int32),
            pltpu.VMEM((NBUF, CHUNK, VE_DIM), jnp.float32),
            pltpu.VMEM((NBUF, CHUNK, VE_DIM), jnp.float32),
            pltpu.SemaphoreType.DMA((NBUF,)),
            pltpu.SemaphoreType.DMA((NBUF,)),
        ],
        compiler_params=pltpu.CompilerParams(use_tc_tiling_on_sc=False),
    )(idx, table)


def _mm_compute(x_ref, w_ref, s_ref, o_ref):
    sc = s_ref[0]
    w = w_ref[...]
    o_ref[0] = (
        jnp.dot(x_ref[:, :VE_DIM], w, preferred_element_type=jnp.float32) * sc
    )
    o_ref[1] = (
        jnp.dot(x_ref[:, VE_DIM:], w, preferred_element_type=jnp.float32) * sc
    )


def _mm_body_first(x_ref, w_ref, s_ref, o_ref):
    _mm_compute(x_ref, w_ref, s_ref, o_ref)


def _mm_body_acc(acc_ref, x_ref, w_ref, s_ref, o_ref):
    del acc_ref
    _mm_compute(x_ref, w_ref, s_ref, o_ref)


C_BLOCKS = HC // R_PAIR


def _tc_project_split(acc, x, w_t, scale, split):
    base = split * C_BLOCKS
    x_spec = pl.BlockSpec((R_PAIR, 2 * VE_DIM), lambda i: (i, 0))
    w_spec = pl.BlockSpec((VE_DIM, MODEL_DIM), lambda i: (0, 0))
    s_spec = pl.BlockSpec(memory_space=pltpu.SMEM)
    out_spec = pl.BlockSpec(
        (2, R_PAIR, MODEL_DIM), lambda i: (0, i + base, 0)
    )
    out_shape = jax.ShapeDtypeStruct((2, HN, MODEL_DIM), jnp.float32)
    if acc is None:
        return pl.pallas_call(
            _mm_body_first,
            grid=(C_BLOCKS,),
            in_specs=[x_spec, w_spec, s_spec],
            out_specs=out_spec,
            out_shape=out_shape,
        )(x, w_t, scale)
    return pl.pallas_call(
        _mm_body_acc,
        grid=(C_BLOCKS,),
        in_specs=[pl.BlockSpec(memory_space=pl.ANY), x_spec, w_spec, s_spec],
        out_specs=out_spec,
        out_shape=out_shape,
        input_output_aliases={0: 0},
    )(acc, x, w_t, scale)


def kernel(token_ids, embed_weight, proj_weight, scale):
    ids = token_ids.astype(jnp.int32).reshape(-1)
    ids = (ids & ~(TBLK - 1)) + 2 * (ids & (HBLK - 1)) + ((ids >> HSHIFT) & 1)
    table_lin = _tc_transpose(embed_weight.T).reshape(VPAD, VE_DIM)
    w_t = proj_weight.T
    sc = scale.reshape(1)
    out = None
    for split in range(NSPLIT):
        paired = _sc_gather(ids, table_lin, split)
        out = _tc_project_split(out, paired, w_t, sc, split)
    return out.reshape(B, L, MODEL_DIM)


# speedup vs baseline: 1.0864x; 1.0272x over previous
import functools

import jax
import jax.numpy as jnp
from jax import lax
from jax.experimental import pallas as pl
from jax.experimental.pallas import tpu as pltpu
from jax.experimental.pallas import tpu_sc as plsc

VOCAB = 1000000
VE_DIM = 64
MODEL_DIM = 128
B = 4096
L = 200
N = B * L
HN = N // 2

NC = 2
NS = 16
NW = NC * NS
NSPLIT = 5
HC = HN // NSPLIT
SPAN = HC // NW
CHUNK = 128
N_CHUNKS = SPAN // CHUNK
NBUF = 4

TBLK = 32768
HBLK = TBLK // 2
HSHIFT = 14
T_GRID = (VOCAB + TBLK - 1) // TBLK
VPAD = T_GRID * TBLK

R_PAIR = 8192
N_BLOCKS = HN // R_PAIR


def _tr_body(x_ref, o_ref):
    eye = jnp.eye(VE_DIM, dtype=jnp.float32)
    o_ref[:, :VE_DIM] = jnp.dot(
        x_ref[:, :HBLK].T, eye, preferred_element_type=jnp.float32
    )
    o_ref[:, VE_DIM:] = jnp.dot(
        x_ref[:, HBLK:].T, eye, preferred_element_type=jnp.float32
    )


def _tc_transpose(wt):
    return pl.pallas_call(
        _tr_body,
        grid=(T_GRID,),
        in_specs=[pl.BlockSpec((VE_DIM, TBLK), lambda i: (0, i))],
        out_specs=pl.BlockSpec((HBLK, 2 * VE_DIM), lambda i: (i, 0)),
        out_shape=jax.ShapeDtypeStruct((T_GRID * HBLK, 2 * VE_DIM), jnp.float32),
        compiler_params=pltpu.CompilerParams(fuse_transposed_lhs_in_matmul=True),
    )(wt)


def _sc_gather_body(coff, idx_hbm, table_hbm, out_hbm, idx_v, bufa, bufb,
                    sema, semb):
    c = lax.axis_index("c")
    s = lax.axis_index("s")
    wid = s * NC + c

    pltpu.sync_copy(idx_hbm.at[pl.ds(coff + wid * SPAN, SPAN)], idx_v.at[0])
    pltpu.sync_copy(idx_hbm.at[pl.ds(HN + coff + wid * SPAN, SPAN)],
                    idx_v.at[1])

    def _fill(j, b):
        pltpu.async_copy(
            table_hbm.at[idx_v.at[0, pl.ds(j * CHUNK, CHUNK)]], bufa.at[b],
            sema.at[b],
        )
        pltpu.async_copy(
            table_hbm.at[idx_v.at[1, pl.ds(j * CHUNK, CHUNK)]], bufb.at[b],
            semb.at[b],
        )

    for b in range(NBUF):
        _fill(b, b)

    row0 = wid * SPAN

    def outer(j0, carry):
        for b in range(NBUF):
            j = j0 * NBUF + b
            pltpu.make_async_copy(
                table_hbm.at[idx_v.at[0, pl.ds(j * CHUNK, CHUNK)]], bufa.at[b],
                sema.at[b],
            ).wait()
            pltpu.make_async_copy(
                table_hbm.at[idx_v.at[1, pl.ds(j * CHUNK, CHUNK)]], bufb.at[b],
                semb.at[b],
            ).wait()
            r = row0 + j * CHUNK
            pltpu.sync_copy(bufa.at[b],
                            out_hbm.at[pl.ds(r, CHUNK), pl.ds(0, VE_DIM)])
            pltpu.sync_copy(bufb.at[b],
                            out_hbm.at[pl.ds(r, CHUNK), pl.ds(VE_DIM, VE_DIM)])

            @pl.when(j + NBUF < N_CHUNKS)
            def _refill(b=b, j=j):
                _fill(j + NBUF, b)

        return carry

    lax.fori_loop(0, N_CHUNKS // NBUF, outer, None)


def _sc_gather(idx, table, split):
    mesh = plsc.VectorSubcoreMesh(core_axis_name="c", subcore_axis_name="s")
    return pl.kernel(
        functools.partial(_sc_gather_body, split * HC),
        out_type=jax.ShapeDtypeStruct((HC, 2 * VE_DIM), jnp.float32),
        name=f"sc_pair_gather_{split}",
        mesh=mesh,
        scratch_types=[
            pltpu.VMEM((2, SPAN), jnp.int32),
            pltpu.VMEM((NBUF, CHUNK, VE_DIM), jnp.float32),
            pltpu.VMEM((NBUF, CHUNK, VE_DIM), jnp.float32),
            pltpu.SemaphoreType.DMA((NBUF,)),
            pltpu.SemaphoreType.DMA((NBUF,)),
        ],
        compiler_params=pltpu.CompilerParams(use_tc_tiling_on_sc=False),
    )(idx, table)


def _mm_compute(x_ref, w_ref, s_ref, o_ref):
    sc = s_ref[0]
    w = w_ref[...]
    o_ref[0] = (
        jnp.dot(x_ref[:, :VE_DIM], w, preferred_element_type=jnp.float32) * sc
    )
    o_ref[1] = (
        jnp.dot(x_ref[:, VE_DIM:], w, preferred_element_type=jnp.float32) * sc
    )


def _mm_body_first(x_ref, w_ref, s_ref, o_ref):
    _mm_compute(x_ref, w_ref, s_ref, o_ref)


def _mm_body_acc(acc_ref, x_ref, w_ref, s_ref, o_ref):
    del acc_ref
    _mm_compute(x_ref, w_ref, s_ref, o_ref)


C_BLOCKS = HC // R_PAIR


def _tc_project_split(acc, x, w_t, scale, split):
    base = split * C_BLOCKS
    x_spec = pl.BlockSpec((R_PAIR, 2 * VE_DIM), lambda i: (i, 0))
    w_spec = pl.BlockSpec((VE_DIM, MODEL_DIM), lambda i: (0, 0))
    s_spec = pl.BlockSpec(memory_space=pltpu.SMEM)
    out_spec = pl.BlockSpec(
        (2, R_PAIR, MODEL_DIM), lambda i: (0, i + base, 0)
    )
    out_shape = jax.ShapeDtypeStruct((2, HN, MODEL_DIM), jnp.float32)
    if acc is None:
        return pl.pallas_call(
            _mm_body_first,
            grid=(C_BLOCKS,),
            in_specs=[x_spec, w_spec, s_spec],
            out_specs=out_spec,
            out_shape=out_shape,
        )(x, w_t, scale)
    return pl.pallas_call(
        _mm_body_acc,
        grid=(C_BLOCKS,),
        in_specs=[pl.BlockSpec(memory_space=pl.ANY), x_spec, w_spec, s_spec],
        out_specs=out_spec,
        out_shape=out_shape,
        input_output_aliases={0: 0},
    )(acc, x, w_t, scale)


def kernel(token_ids, embed_weight, proj_weight, scale):
    ids = token_ids.astype(jnp.int32).reshape(-1)
    ids = (ids & ~(TBLK - 1)) + 2 * (ids & (HBLK - 1)) + ((ids >> HSHIFT) & 1)
    table_lin = _tc_transpose(embed_weight.T).reshape(VPAD, VE_DIM)
    w_t = proj_weight.T
    sc = scale.reshape(1)
    out = None
    for split in range(NSPLIT):
        paired = _sc_gather(ids, table_lin, split)
        out = _tc_project_split(out, paired, w_t, sc, split)
    return out.reshape(B, L, MODEL_DIM)


# speedup vs baseline: 1.0865x; 1.0000x over previous
import functools

import jax
import jax.numpy as jnp
from jax import lax
from jax.experimental import pallas as pl
from jax.experimental.pallas import tpu as pltpu
from jax.experimental.pallas import tpu_sc as plsc

VOCAB = 1000000
VE_DIM = 64
MODEL_DIM = 128
B = 4096
L = 200
N = B * L
HN = N // 2

NC = 2
NS = 16
NW = NC * NS
NSPLIT = 5
HC = HN // NSPLIT
SPAN = HC // NW
CHUNK = 128
N_CHUNKS = SPAN // CHUNK
NBUF = 4

TBLK = 32768
HBLK = TBLK // 2
HSHIFT = 14
T_GRID = (VOCAB + TBLK - 1) // TBLK
VPAD = T_GRID * TBLK

R_PAIR = 8192
N_BLOCKS = HN // R_PAIR


def _tr_body(x_ref, o_ref):
    eye = jnp.eye(VE_DIM, dtype=jnp.float32)
    o_ref[:, :VE_DIM] = jnp.dot(
        x_ref[:, :HBLK].T, eye, preferred_element_type=jnp.float32
    )
    o_ref[:, VE_DIM:] = jnp.dot(
        x_ref[:, HBLK:].T, eye, preferred_element_type=jnp.float32
    )


def _tc_transpose(wt):
    return pl.pallas_call(
        _tr_body,
        grid=(T_GRID,),
        in_specs=[pl.BlockSpec((VE_DIM, TBLK), lambda i: (0, i))],
        out_specs=pl.BlockSpec((HBLK, 2 * VE_DIM), lambda i: (i, 0)),
        out_shape=jax.ShapeDtypeStruct((T_GRID * HBLK, 2 * VE_DIM), jnp.float32),
        compiler_params=pltpu.CompilerParams(
            fuse_transposed_lhs_in_matmul=True,
            vmem_limit_bytes=100 * 1024 * 1024,
        ),
    )(wt)


def _sc_gather_body(coff, idx_hbm, table_hbm, out_hbm, idx_v, bufa, bufb,
                    sema, semb):
    c = lax.axis_index("c")
    s = lax.axis_index("s")
    wid = s * NC + c

    pltpu.sync_copy(idx_hbm.at[pl.ds(coff + wid * SPAN, SPAN)], idx_v.at[0])
    pltpu.sync_copy(idx_hbm.at[pl.ds(HN + coff + wid * SPAN, SPAN)],
                    idx_v.at[1])

    def _fill(j, b):
        pltpu.async_copy(
            table_hbm.at[idx_v.at[0, pl.ds(j * CHUNK, CHUNK)]], bufa.at[b],
            sema.at[b],
        )
        pltpu.async_copy(
            table_hbm.at[idx_v.at[1, pl.ds(j * CHUNK, CHUNK)]], bufb.at[b],
            semb.at[b],
        )

    for b in range(NBUF):
        _fill(b, b)

    row0 = wid * SPAN

    def outer(j0, carry):
        for b in range(NBUF):
            j = j0 * NBUF + b
            pltpu.make_async_copy(
                table_hbm.at[idx_v.at[0, pl.ds(j * CHUNK, CHUNK)]], bufa.at[b],
                sema.at[b],
            ).wait()
            pltpu.make_async_copy(
                table_hbm.at[idx_v.at[1, pl.ds(j * CHUNK, CHUNK)]], bufb.at[b],
                semb.at[b],
            ).wait()
            r = row0 + j * CHUNK
            pltpu.sync_copy(bufa.at[b],
                            out_hbm.at[pl.ds(r, CHUNK), pl.ds(0, VE_DIM)])
            pltpu.sync_copy(bufb.at[b],
                            out_hbm.at[pl.ds(r, CHUNK), pl.ds(VE_DIM, VE_DIM)])

            @pl.when(j + NBUF < N_CHUNKS)
            def _refill(b=b, j=j):
                _fill(j + NBUF, b)

        return carry

    lax.fori_loop(0, N_CHUNKS // NBUF, outer, None)


def _sc_gather(idx, table, split):
    mesh = plsc.VectorSubcoreMesh(core_axis_name="c", subcore_axis_name="s")
    return pl.kernel(
        functools.partial(_sc_gather_body, split * HC),
        out_type=jax.ShapeDtypeStruct((HC, 2 * VE_DIM), jnp.float32),
        name=f"sc_pair_gather_{split}",
        mesh=mesh,
        scratch_types=[
            pltpu.VMEM((2, SPAN), jnp.int32),
            pltpu.VMEM((NBUF, CHUNK, VE_DIM), jnp.float32),
            pltpu.VMEM((NBUF, CHUNK, VE_DIM), jnp.float32),
            pltpu.SemaphoreType.DMA((NBUF,)),
            pltpu.SemaphoreType.DMA((NBUF,)),
        ],
        compiler_params=pltpu.CompilerParams(use_tc_tiling_on_sc=False),
    )(idx, table)


def _mm_compute(x_ref, w_ref, s_ref, o_ref):
    sc = s_ref[0]
    w = w_ref[...]
    o_ref[0] = (
        jnp.dot(x_ref[:, :VE_DIM], w, preferred_element_type=jnp.float32) * sc
    )
    o_ref[1] = (
        jnp.dot(x_ref[:, VE_DIM:], w, preferred_element_type=jnp.float32) * sc
    )


def _mm_body_first(x_ref, w_ref, s_ref, o_ref):
    _mm_compute(x_ref, w_ref, s_ref, o_ref)


def _mm_body_acc(acc_ref, x_ref, w_ref, s_ref, o_ref):
    del acc_ref
    _mm_compute(x_ref, w_ref, s_ref, o_ref)


C_BLOCKS = HC // R_PAIR


def _tc_project_split(acc, x, w_t, scale, split):
    base = split * C_BLOCKS
    x_spec = pl.BlockSpec((R_PAIR, 2 * VE_DIM), lambda i: (i, 0))
    w_spec = pl.BlockSpec((VE_DIM, MODEL_DIM), lambda i: (0, 0))
    s_spec = pl.BlockSpec(memory_space=pltpu.SMEM)
    out_spec = pl.BlockSpec(
        (2, R_PAIR, MODEL_DIM), lambda i: (0, i + base, 0)
    )
    out_shape = jax.ShapeDtypeStruct((2, HN, MODEL_DIM), jnp.float32)
    if acc is None:
        return pl.pallas_call(
            _mm_body_first,
            grid=(C_BLOCKS,),
            in_specs=[x_spec, w_spec, s_spec],
            out_specs=out_spec,
            out_shape=out_shape,
        )(x, w_t, scale)
    return pl.pallas_call(
        _mm_body_acc,
        grid=(C_BLOCKS,),
        in_specs=[pl.BlockSpec(memory_space=pl.ANY), x_spec, w_spec, s_spec],
        out_specs=out_spec,
        out_shape=out_shape,
        input_output_aliases={0: 0},
    )(acc, x, w_t, scale)


def kernel(token_ids, embed_weight, proj_weight, scale):
    ids = token_ids.astype(jnp.int32).reshape(-1)
    ids = (ids & ~(TBLK - 1)) + 2 * (ids & (HBLK - 1)) + ((ids >> HSHIFT) & 1)
    table_lin = _tc_transpose(embed_weight.T).reshape(VPAD, VE_DIM)
    w_t = proj_weight.T
    sc = scale.reshape(1)
    out = None
    for split in range(NSPLIT):
        paired = _sc_gather(ids, table_lin, split)
        out = _tc_project_split(out, paired, w_t, sc, split)
    return out.reshape(B, L, MODEL_DIM)
